# baseline (speedup 1.0000x reference)
import functools

import jax
import jax.numpy as jnp
from jax import lax
from jax.experimental import pallas as pl
from jax.experimental.pallas import tpu as pltpu
from jax.experimental.pallas import tpu_sc as plsc

N = 10000
D = 128
DFF = 256
OUT = 10
G = 128
E = 320000
EPS_MSG = 1e-7
BN_EPS = 1e-5

NC = 2
NS = 16
CHUNK = 120
GRP = 6
CHG = 28
CH = CHG * GRP
EPAD = NS * CH * CHUNK
ACC_ROWS = 10128
RPT = 632
RPT_LAST = 648

BN_BLK = 1000
NB = N // BN_BLK


def _sc_edge_sum(table, src2, dst3):
    mesh = plsc.VectorSubcoreMesh(
        core_axis_name="c", subcore_axis_name="s", num_cores=NC, num_subcores=NS
    )

    @functools.partial(
        pl.kernel,
        out_type=jax.ShapeDtypeStruct((NC, ACC_ROWS, D), jnp.float32),
        mesh=mesh,
        scratch_types=[
            pltpu.VMEM_SHARED((ACC_ROWS, D), jnp.float32),
            pltpu.VMEM((GRP, CHUNK), jnp.int32),
            pltpu.VMEM((GRP, CHUNK), jnp.int32),
            pltpu.VMEM((3, CHUNK, D), jnp.float32),
            pltpu.SemaphoreType.DMA,
            pltpu.SemaphoreType.DMA,
            pltpu.SemaphoreType.DMA,
            pltpu.SemaphoreType.DMA,
        ],
    )
    def k(table_h, src_h, dst_h, out_h, acc_sh, src_v, dst_v, rows_v,
          sem_0, sem_1, sem_2, sem_d):
        c = lax.axis_index("c")
        s = lax.axis_index("s")
        sems = (sem_0, sem_1, sem_2)
        base = s * RPT

        pltpu.sync_copy(src_h.at[c, s, 0], src_v)
        pltpu.async_copy(table_h.at[src_v.at[0]], rows_v.at[0], sems[0])
        pltpu.async_copy(table_h.at[src_v.at[1]], rows_v.at[1], sems[1])
        pltpu.async_copy(dst_h.at[s, 0], dst_v, sem_d)

        zeros16 = jnp.zeros((16,), jnp.float32)

        def zrow(r, carry):
            for kk in range(D // 16):
                rows_v[2, r, pl.ds(kk * 16, 16)] = zeros16
            return carry

        lax.fori_loop(0, CHUNK, zrow, None)
        for j in range(5):
            pltpu.sync_copy(rows_v.at[2],
                            acc_sh.at[pl.ds(base + j * CHUNK, CHUNK)])

        @pl.when(s < NS - 1)
        def _():
            pltpu.sync_copy(rows_v.at[2, pl.ds(0, 32)],
                            acc_sh.at[pl.ds(base + 5 * CHUNK, 32)])

        @pl.when(s == NS - 1)
        def _():
            pltpu.sync_copy(rows_v.at[2, pl.ds(0, 48)],
                            acc_sh.at[pl.ds(base + 5 * CHUNK, 48)])

        plsc.subcore_barrier()

        def group(g, carry):
            for k in range(GRP):
                p = k % 3
                if k < GRP - 2:
                    pltpu.async_copy(table_h.at[src_v.at[k + 2]],
                                     rows_v.at[(k + 2) % 3], sems[(k + 2) % 3])
                elif k == GRP - 2:
                    pltpu.sync_copy(src_h.at[c, s, g + 1], src_v)
                    pltpu.async_copy(table_h.at[src_v.at[0]], rows_v.at[0],
                                     sems[0])
                else:
                    pltpu.async_copy(table_h.at[src_v.at[1]], rows_v.at[1],
                                     sems[1])
                pltpu.make_async_copy(table_h.at[src_v.at[k]], rows_v.at[p],
                                      sems[p]).wait()
                if k == 0:
                    pltpu.make_async_copy(dst_h.at[s, g], dst_v, sem_d).wait()
                pltpu.sync_copy(rows_v.at[p], acc_sh.at[dst_v.at[k]],
                                add=True)
                if k == GRP - 1:
                    pltpu.async_copy(dst_h.at[s, g + 1], dst_v, sem_d)
            return carry

        lax.fori_loop(0, CHG, group, None)

        pltpu.make_async_copy(table_h.at[src_v.at[0]], rows_v.at[0],
                              sems[0]).wait()
        pltpu.make_async_copy(table_h.at[src_v.at[1]], rows_v.at[1],
                              sems[1]).wait()
        pltpu.make_async_copy(dst_h.at[s, CHG], dst_v, sem_d).wait()
        plsc.subcore_barrier()

        for j in range(5):
            pltpu.sync_copy(acc_sh.at[pl.ds(base + j * CHUNK, CHUNK)],
                            rows_v.at[0])
            pltpu.sync_copy(rows_v.at[0],
                            out_h.at[c, pl.ds(base + j * CHUNK, CHUNK)])

        @pl.when(s < NS - 1)
        def _():
            pltpu.sync_copy(acc_sh.at[pl.ds(base + 5 * CHUNK, 32)],
                            rows_v.at[0, pl.ds(0, 32)])
            pltpu.sync_copy(rows_v.at[0, pl.ds(0, 32)],
                            out_h.at[c, pl.ds(base + 5 * CHUNK, 32)])

        @pl.when(s == NS - 1)
        def _():
            pltpu.sync_copy(acc_sh.at[pl.ds(base + 5 * CHUNK, 48)],
                            rows_v.at[0, pl.ds(0, 48)])
            pltpu.sync_copy(rows_v.at[0, pl.ds(0, 48)],
                            out_h.at[c, pl.ds(base + 5 * CHUNK, 48)])

    return k(table, src2, dst3)


def _colmax(h):
    def body(h_ref, o_ref):
        i = pl.program_id(0)
        m = jnp.max(jax.nn.relu(h_ref[...]), axis=0, keepdims=True)

        @pl.when(i == 0)
        def _():
            o_ref[...] = m

        @pl.when(i > 0)
        def _():
            o_ref[...] = jnp.maximum(o_ref[...], m)

    return pl.pallas_call(
        body,
        grid=(NB,),
        in_specs=[pl.BlockSpec((BN_BLK, D), lambda i: (i, 0))],
        out_specs=pl.BlockSpec((1, D), lambda i: (0, 0)),
        out_shape=jax.ShapeDtypeStruct((1, D), jnp.float32),
    )(h)


def _prep(h, cmax):
    def body(h_ref, c_ref, o_ref):
        m = jax.nn.relu(h_ref[...]) + EPS_MSG
        p = jnp.exp(m - (c_ref[...] + EPS_MSG))
        o_ref[0] = p
        o_ref[1] = m * p

    return pl.pallas_call(
        body,
        grid=(NB,),
        in_specs=[
            pl.BlockSpec((BN_BLK, D), lambda i: (i, 0)),
            pl.BlockSpec((1, D), lambda i: (0, 0)),
        ],
        out_specs=pl.BlockSpec((2, BN_BLK, D), lambda i: (0, i, 0)),
        out_shape=jax.ShapeDtypeStruct((2, N, D), jnp.float32),
    )(h, cmax)


def _mlp1(S, h, Wa, ba):
    def body(s_ref, h_ref, wa_ref, ba_ref, t_ref, st_ref):
        i = pl.program_id(0)
        den = s_ref[0]
        num = s_ref[1]
        agg = num / (den + 1e-30)
        out = agg + h_ref[...]
        t = jnp.dot(out, wa_ref[...], preferred_element_type=jnp.float32)
        t = t + ba_ref[...]
        t_ref[...] = t

        @pl.when(i == 0)
        def _():
            st_ref[...] = jnp.zeros_like(st_ref)

        st_ref[...] += jnp.concatenate(
            [jnp.sum(t, axis=0, keepdims=True),
             jnp.sum(t * t, axis=0, keepdims=True)], axis=0)

    return pl.pallas_call(
        body,
        grid=(NB,),
        in_specs=[
            pl.BlockSpec((2, BN_BLK, D), lambda i: (0, i, 0)),
            pl.BlockSpec((BN_BLK, D), lambda i: (i, 0)),
            pl.BlockSpec((D, DFF), lambda i: (0, 0)),
            pl.BlockSpec((1, DFF), lambda i: (0, 0)),
        ],
        out_specs=[
            pl.BlockSpec((BN_BLK, DFF), lambda i: (i, 0)),
            pl.BlockSpec((2, DFF), lambda i: (0, 0)),
        ],
        out_shape=[
            jax.ShapeDtypeStruct((N, DFF), jnp.float32),
            jax.ShapeDtypeStruct((2, DFF), jnp.float32),
        ],
    )(S, h, Wa, ba)


def _mlp2(t, st, bnw, bnb, Wb, bb):
    def body(t_ref, st_ref, w_ref, b_ref, wb_ref, bb_ref, o_ref, cm_ref):
        i = pl.program_id(0)
        mu = st_ref[0:1] * (1.0 / N)
        var = jnp.maximum(st_ref[1:2] * (1.0 / N) - mu * mu, 0.0)
        inv = lax.rsqrt(var + BN_EPS)
        scale = inv * w_ref[...]
        shift = b_ref[...] - mu * scale
        hmid = jax.nn.relu(t_ref[...] * scale + shift)
        y = jnp.dot(hmid, wb_ref[...], preferred_element_type=jnp.float32)
        y = jax.nn.relu(y + bb_ref[...])
        o_ref[...] = y
        m = jnp.max(y, axis=0, keepdims=True)

        @pl.when(i == 0)
        def _():
            cm_ref[...] = m

        @pl.when(i > 0)
        def _():
            cm_ref[...] = jnp.maximum(cm_ref[...], m)

    return pl.pallas_call(
        body,
        grid=(NB,),
        in_specs=[
            pl.BlockSpec((BN_BLK, DFF), lambda i: (i, 0)),
            pl.BlockSpec((2, DFF), lambda i: (0, 0)),
            pl.BlockSpec((1, DFF), lambda i: (0, 0)),
            pl.BlockSpec((1, DFF), lambda i: (0, 0)),
            pl.BlockSpec((DFF, D), lambda i: (0, 0)),
            pl.BlockSpec((1, D), lambda i: (0, 0)),
        ],
        out_specs=[
            pl.BlockSpec((BN_BLK, D), lambda i: (i, 0)),
            pl.BlockSpec((1, D), lambda i: (0, 0)),
        ],
        out_shape=[
            jax.ShapeDtypeStruct((N, D), jnp.float32),
            jax.ShapeDtypeStruct((1, D), jnp.float32),
        ],
    )(t, st, bnw, bnb, Wb, bb)


def _pool_head(h, batch3, Wd, bd):
    def body(h_ref, b_ref, wd_ref, bd_ref, o_ref, sums, cnts):
        i = pl.program_id(0)

        @pl.when(i == 0)
        def _():
            sums[...] = jnp.zeros_like(sums)
            cnts[...] = jnp.zeros_like(cnts)

        ids = b_ref[...]
        iota = lax.broadcasted_iota(jnp.int32, (1, G, BN_BLK), 1)
        oht = (ids == iota).astype(jnp.float32)[0]
        sums[...] += lax.dot_general(
            oht, h_ref[...], (((1,), (0,)), ((), ())),
            preferred_element_type=jnp.float32)
        cnts[...] += lax.dot_general(
            oht, jnp.ones((BN_BLK, 1), jnp.float32), (((1,), (0,)), ((), ())),
            preferred_element_type=jnp.float32)

        @pl.when(i == NB - 1)
        def _():
            pooled = sums[...] / jnp.maximum(cnts[...], 1.0)
            logits = jnp.dot(pooled, wd_ref[...],
                             preferred_element_type=jnp.float32) + bd_ref[...]
            o_ref[...] = jax.nn.sigmoid(logits)

    return pl.pallas_call(
        body,
        grid=(NB,),
        in_specs=[
            pl.BlockSpec((BN_BLK, D), lambda i: (i, 0)),
            pl.BlockSpec((1, 1, BN_BLK), lambda i: (i, 0, 0)),
            pl.BlockSpec((D, OUT), lambda i: (0, 0)),
            pl.BlockSpec((1, OUT), lambda i: (0, 0)),
        ],
        out_specs=pl.BlockSpec((G, OUT), lambda i: (0, 0)),
        out_shape=jax.ShapeDtypeStruct((G, OUT), jnp.float32),
        scratch_shapes=[
            pltpu.VMEM((G, D), jnp.float32),
            pltpu.VMEM((G, 1), jnp.float32),
        ],
    )(h, batch3, Wd, bd)


def kernel(x, edge_index, batch,
           W1a, b1a, bn1w, bn1b, W1b, b1b,
           W2a, b2a, bn2w, bn2b, W2b, b2b,
           W3a, b3a, bn3w, bn3b, W3b, b3b,
           Wd, bd):
    src = edge_index[0]
    dst = edge_index[1]
    pad = EPAD - E
    ar = jnp.arange(pad, dtype=jnp.int32)
    gpt = GRP * CHUNK
    src_p = jnp.concatenate([src, (ar * 67) % N]).reshape(NS, CHG, GRP, CHUNK)
    dst_p = jnp.concatenate([dst, N + (ar % 128)]).reshape(NS, CHG, GRP, CHUNK)
    xtra = (jnp.arange(NS * gpt, dtype=jnp.int32) * 131) % N
    xtra = xtra.reshape(NS, 1, GRP, CHUNK)
    src_p = jnp.concatenate([src_p, xtra], axis=1)
    dst_p = jnp.concatenate([dst_p, xtra], axis=1)
    dst3 = dst_p
    src2 = jnp.stack([src_p, src_p + N])

    batch3 = batch.reshape(NB, 1, BN_BLK)

    params = [
        (W1a, b1a, bn1w, bn1b, W1b, b1b),
        (W2a, b2a, bn2w, bn2b, W2b, b2b),
        (W3a, b3a, bn3w, bn3b, W3b, b3b),
    ]
    h = x
    cmax = _colmax(x)
    for Wa, ba, bnw, bnb, Wb, bb in params:
        pq = _prep(h, cmax)
        S = _sc_edge_sum(pq.reshape(2 * N, D), src2, dst3)
        t, st = _mlp1(S, h, Wa, ba.reshape(1, DFF))
        h, cmax = _mlp2(t, st, bnw.reshape(1, DFF), bnb.reshape(1, DFF),
                        Wb, bb.reshape(1, D))
    return _pool_head(h, batch3, Wd, bd.reshape(1, OUT))


# speedup vs baseline: 1.0422x; 1.0422x over previous
import functools

import jax
import jax.numpy as jnp
from jax import lax
from jax.experimental import pallas as pl
from jax.experimental.pallas import tpu as pltpu
from jax.experimental.pallas import tpu_sc as plsc

N = 10000
D = 128
DFF = 256
OUT = 10
G = 128
E = 320000
EPS_MSG = 1e-7
BN_EPS = 1e-5

NC = 2
NS = 16
CHUNK = 120
GRP = 6
CHG = 28
CH = CHG * GRP
EPAD = NS * CH * CHUNK
ACC_ROWS = 10128
RPT = 632
RPT_LAST = 648

BN_BLK = 1000
NB = N // BN_BLK


def _sc_edge_sum(table, src2, dst3):
    mesh = plsc.VectorSubcoreMesh(
        core_axis_name="c", subcore_axis_name="s", num_cores=NC, num_subcores=NS
    )

    @functools.partial(
        pl.kernel,
        out_type=jax.ShapeDtypeStruct((NC, ACC_ROWS, D), jnp.float32),
        mesh=mesh,
        scratch_types=[
            pltpu.VMEM_SHARED((ACC_ROWS, D), jnp.float32),
            pltpu.VMEM((GRP, CHUNK), jnp.int32),
            pltpu.VMEM((GRP, CHUNK), jnp.int32),
            pltpu.VMEM((3, CHUNK, D), jnp.float32),
            pltpu.SemaphoreType.DMA,
            pltpu.SemaphoreType.DMA,
            pltpu.SemaphoreType.DMA,
            pltpu.SemaphoreType.DMA,
        ],
    )
    def k(table_h, src_h, dst_h, out_h, acc_sh, src_v, dst_v, rows_v,
          sem_0, sem_1, sem_2, sem_d):
        c = lax.axis_index("c")
        s = lax.axis_index("s")
        sems = (sem_0, sem_1, sem_2)
        base = s * RPT

        pltpu.sync_copy(src_h.at[c, s, 0], src_v)
        pltpu.async_copy(table_h.at[src_v.at[0]], rows_v.at[0], sems[0])
        pltpu.async_copy(table_h.at[src_v.at[1]], rows_v.at[1], sems[1])
        pltpu.async_copy(dst_h.at[s, 0], dst_v, sem_d)

        zeros16 = jnp.zeros((16,), jnp.float32)

        def zrow(r, carry):
            for kk in range(D // 16):
                rows_v[2, r, pl.ds(kk * 16, 16)] = zeros16
            return carry

        lax.fori_loop(0, CHUNK, zrow, None)
        for j in range(5):
            pltpu.sync_copy(rows_v.at[2],
                            acc_sh.at[pl.ds(base + j * CHUNK, CHUNK)])

        @pl.when(s < NS - 1)
        def _():
            pltpu.sync_copy(rows_v.at[2, pl.ds(0, 32)],
                            acc_sh.at[pl.ds(base + 5 * CHUNK, 32)])

        @pl.when(s == NS - 1)
        def _():
            pltpu.sync_copy(rows_v.at[2, pl.ds(0, 48)],
                            acc_sh.at[pl.ds(base + 5 * CHUNK, 48)])

        plsc.subcore_barrier()

        def group(g, carry):
            for k in range(GRP):
                p = k % 3
                if k < GRP - 2:
                    pltpu.async_copy(table_h.at[src_v.at[k + 2]],
                                     rows_v.at[(k + 2) % 3], sems[(k + 2) % 3])
                elif k == GRP - 2:
                    pltpu.sync_copy(src_h.at[c, s, g + 1], src_v)
                    pltpu.async_copy(table_h.at[src_v.at[0]], rows_v.at[0],
                                     sems[0])
                else:
                    pltpu.async_copy(table_h.at[src_v.at[1]], rows_v.at[1],
                                     sems[1])
                pltpu.make_async_copy(table_h.at[src_v.at[k]], rows_v.at[p],
                                      sems[p]).wait()
                if k == 0:
                    pltpu.make_async_copy(dst_h.at[s, g], dst_v, sem_d).wait()
                pltpu.sync_copy(rows_v.at[p], acc_sh.at[dst_v.at[k]],
                                add=True)
                if k == GRP - 1:
                    pltpu.async_copy(dst_h.at[s, g + 1], dst_v, sem_d)
            return carry

        lax.fori_loop(0, CHG, group, None)

        pltpu.make_async_copy(table_h.at[src_v.at[0]], rows_v.at[0],
                              sems[0]).wait()
        pltpu.make_async_copy(table_h.at[src_v.at[1]], rows_v.at[1],
                              sems[1]).wait()
        pltpu.make_async_copy(dst_h.at[s, CHG], dst_v, sem_d).wait()
        plsc.subcore_barrier()

        for j in range(5):
            pltpu.sync_copy(acc_sh.at[pl.ds(base + j * CHUNK, CHUNK)],
                            rows_v.at[0])
            pltpu.sync_copy(rows_v.at[0],
                            out_h.at[c, pl.ds(base + j * CHUNK, CHUNK)])

        @pl.when(s < NS - 1)
        def _():
            pltpu.sync_copy(acc_sh.at[pl.ds(base + 5 * CHUNK, 32)],
                            rows_v.at[0, pl.ds(0, 32)])
            pltpu.sync_copy(rows_v.at[0, pl.ds(0, 32)],
                            out_h.at[c, pl.ds(base + 5 * CHUNK, 32)])

        @pl.when(s == NS - 1)
        def _():
            pltpu.sync_copy(acc_sh.at[pl.ds(base + 5 * CHUNK, 48)],
                            rows_v.at[0, pl.ds(0, 48)])
            pltpu.sync_copy(rows_v.at[0, pl.ds(0, 48)],
                            out_h.at[c, pl.ds(base + 5 * CHUNK, 48)])

    return k(table, src2, dst3)


def _prep_block(h_blk, cm):
    m = jax.nn.relu(h_blk) + EPS_MSG
    p = jnp.exp(m - (cm + EPS_MSG))
    return p, m * p


def _first_prep(x):
    def body(x_ref, o_ref, cm_scr):
        i = pl.program_id(0)
        ph = i // NB

        @pl.when(ph == 0)
        def _():
            m = jnp.max(jax.nn.relu(x_ref[...]), axis=0, keepdims=True)

            @pl.when(i == 0)
            def _():
                cm_scr[...] = m

            @pl.when(i > 0)
            def _():
                cm_scr[...] = jnp.maximum(cm_scr[...], m)

        @pl.when(ph == 1)
        def _():
            p, q = _prep_block(x_ref[...], cm_scr[...])
            o_ref[0] = p
            o_ref[1] = q

    return pl.pallas_call(
        body,
        grid=(2 * NB,),
        in_specs=[pl.BlockSpec((BN_BLK, D), lambda i: (i % NB, 0))],
        out_specs=pl.BlockSpec(
            (2, BN_BLK, D),
            lambda i: (0, jnp.where(i // NB == 1, i % NB, 0), 0)),
        out_shape=jax.ShapeDtypeStruct((2, N, D), jnp.float32),
        scratch_shapes=[pltpu.VMEM((1, D), jnp.float32)],
    )(x)


def _mlp_common(s_ref, h_ref, wa_ref, ba_ref, w_ref, b_ref, wb_ref, bb_ref,
                t_scr, st_scr, y_scr, blk, ph):
    @pl.when(ph == 0)
    def _():
        den = s_ref[0]
        num = s_ref[1]
        out = num / (den + 1e-30) + h_ref[...]
        t = jnp.dot(out, wa_ref[...], preferred_element_type=jnp.float32)
        t = t + ba_ref[...]
        t_scr[blk] = t

        @pl.when(blk == 0)
        def _():
            st_scr[...] = jnp.zeros_like(st_scr)

        st_scr[...] += jnp.concatenate(
            [jnp.sum(t, axis=0, keepdims=True),
             jnp.sum(t * t, axis=0, keepdims=True)], axis=0)

    @pl.when(ph == 1)
    def _():
        mu = st_scr[0:1] * (1.0 / N)
        var = jnp.maximum(st_scr[1:2] * (1.0 / N) - mu * mu, 0.0)
        inv = lax.rsqrt(var + BN_EPS)
        scale = inv * w_ref[...]
        shift = b_ref[...] - mu * scale
        hmid = jax.nn.relu(t_scr[blk] * scale + shift)
        y = jnp.dot(hmid, wb_ref[...], preferred_element_type=jnp.float32)
        y_scr[blk] = jax.nn.relu(y + bb_ref[...])


def _mid_layer(S, h, Wa, ba, bnw, bnb, Wb, bb):
    def body(s_ref, h_ref, wa_ref, ba_ref, w_ref, b_ref, wb_ref, bb_ref,
             pq_ref, ho_ref, t_scr, st_scr, y_scr, cm_scr):
        i = pl.program_id(0)
        ph = i // NB
        blk = i % NB
        _mlp_common(s_ref, h_ref, wa_ref, ba_ref, w_ref, b_ref, wb_ref,
                    bb_ref, t_scr, st_scr, y_scr, blk, ph)

        @pl.when(ph == 1)
        def _():
            y = y_scr[blk]
            ho_ref[...] = y
            m = jnp.max(y, axis=0, keepdims=True)

            @pl.when(blk == 0)
            def _():
                cm_scr[...] = m

            @pl.when(blk > 0)
            def _():
                cm_scr[...] = jnp.maximum(cm_scr[...], m)

        @pl.when(ph == 2)
        def _():
            p, q = _prep_block(y_scr[blk], cm_scr[...])
            pq_ref[0] = p
            pq_ref[1] = q

    return pl.pallas_call(
        body,
        grid=(3 * NB,),
        in_specs=[
            pl.BlockSpec((2, BN_BLK, D),
                         lambda i: (0, jnp.where(i // NB == 0, i % NB, 0), 0)),
            pl.BlockSpec((BN_BLK, D),
                         lambda i: (jnp.where(i // NB == 0, i % NB, 0), 0)),
            pl.BlockSpec((D, DFF), lambda i: (0, 0)),
            pl.BlockSpec((1, DFF), lambda i: (0, 0)),
            pl.BlockSpec((1, DFF), lambda i: (0, 0)),
            pl.BlockSpec((1, DFF), lambda i: (0, 0)),
            pl.BlockSpec((DFF, D), lambda i: (0, 0)),
            pl.BlockSpec((1, D), lambda i: (0, 0)),
        ],
        out_specs=[
            pl.BlockSpec(
                (2, BN_BLK, D),
                lambda i: (0, jnp.where(i // NB == 2, i % NB, 0), 0)),
            pl.BlockSpec(
                (BN_BLK, D),
                lambda i: (jnp.where(i // NB == 1, i % NB,
                                     jnp.where(i // NB == 2, NB - 1, 0)), 0)),
        ],
        out_shape=[
            jax.ShapeDtypeStruct((2, N, D), jnp.float32),
            jax.ShapeDtypeStruct((N, D), jnp.float32),
        ],
        scratch_shapes=[
            pltpu.VMEM((NB, BN_BLK, DFF), jnp.float32),
            pltpu.VMEM((2, DFF), jnp.float32),
            pltpu.VMEM((NB, BN_BLK, D), jnp.float32),
            pltpu.VMEM((1, D), jnp.float32),
        ],
    )(S, h, Wa, ba, bnw, bnb, Wb, bb)


def _last_layer(S, h, Wa, ba, bnw, bnb, Wb, bb, batch3, Wd, bd):
    def body(s_ref, h_ref, wa_ref, ba_ref, w_ref, b_ref, wb_ref, bb_ref,
             bt_ref, wd_ref, bd_ref, o_ref, t_scr, st_scr, y_scr, sums, cnts):
        i = pl.program_id(0)
        ph = i // NB
        blk = i % NB
        _mlp_common(s_ref, h_ref, wa_ref, ba_ref, w_ref, b_ref, wb_ref,
                    bb_ref, t_scr, st_scr, y_scr, blk, ph)

        @pl.when(ph == 2)
        def _():
            @pl.when(blk == 0)
            def _():
                sums[...] = jnp.zeros_like(sums)
                cnts[...] = jnp.zeros_like(cnts)

            ids = bt_ref[...]
            iota = lax.broadcasted_iota(jnp.int32, (1, G, BN_BLK), 1)
            oht = (ids == iota).astype(jnp.float32)[0]
            sums[...] += lax.dot_general(
                oht, y_scr[blk], (((1,), (0,)), ((), ())),
                preferred_element_type=jnp.float32)
            cnts[...] += lax.dot_general(
                oht, jnp.ones((BN_BLK, 1), jnp.float32),
                (((1,), (0,)), ((), ())), preferred_element_type=jnp.float32)

            @pl.when(blk == NB - 1)
            def _():
                pooled = sums[...] / jnp.maximum(cnts[...], 1.0)
                logits = jnp.dot(pooled, wd_ref[...],
                                 preferred_element_type=jnp.float32)
                o_ref[...] = jax.nn.sigmoid(logits + bd_ref[...])

    return pl.pallas_call(
        body,
        grid=(3 * NB,),
        in_specs=[
            pl.BlockSpec((2, BN_BLK, D),
                         lambda i: (0, jnp.where(i // NB == 0, i % NB, 0), 0)),
            pl.BlockSpec((BN_BLK, D),
                         lambda i: (jnp.where(i // NB == 0, i % NB, 0), 0)),
            pl.BlockSpec((D, DFF), lambda i: (0, 0)),
            pl.BlockSpec((1, DFF), lambda i: (0, 0)),
            pl.BlockSpec((1, DFF), lambda i: (0, 0)),
            pl.BlockSpec((1, DFF), lambda i: (0, 0)),
            pl.BlockSpec((DFF, D), lambda i: (0, 0)),
            pl.BlockSpec((1, D), lambda i: (0, 0)),
            pl.BlockSpec((1, 1, BN_BLK),
                         lambda i: (jnp.where(i // NB == 2, i % NB, 0), 0, 0)),
            pl.BlockSpec((D, OUT), lambda i: (0, 0)),
            pl.BlockSpec((1, OUT), lambda i: (0, 0)),
        ],
        out_specs=pl.BlockSpec((G, OUT), lambda i: (0, 0)),
        out_shape=jax.ShapeDtypeStruct((G, OUT), jnp.float32),
        scratch_shapes=[
            pltpu.VMEM((NB, BN_BLK, DFF), jnp.float32),
            pltpu.VMEM((2, DFF), jnp.float32),
            pltpu.VMEM((NB, BN_BLK, D), jnp.float32),
            pltpu.VMEM((G, D), jnp.float32),
            pltpu.VMEM((G, 1), jnp.float32),
        ],
    )(S, h, Wa, ba, bnw, bnb, Wb, bb, batch3, Wd, bd)


def kernel(x, edge_index, batch,
           W1a, b1a, bn1w, bn1b, W1b, b1b,
           W2a, b2a, bn2w, bn2b, W2b, b2b,
           W3a, b3a, bn3w, bn3b, W3b, b3b,
           Wd, bd):
    src = edge_index[0]
    dst = edge_index[1]
    pad = EPAD - E
    ar = jnp.arange(pad, dtype=jnp.int32)
    gpt = GRP * CHUNK
    src_p = jnp.concatenate([src, (ar * 67) % N]).reshape(NS, CHG, GRP, CHUNK)
    dst_p = jnp.concatenate([dst, N + (ar % 128)]).reshape(NS, CHG, GRP, CHUNK)
    xtra = (jnp.arange(NS * gpt, dtype=jnp.int32) * 131) % N
    xtra = xtra.reshape(NS, 1, GRP, CHUNK)
    src_p = jnp.concatenate([src_p, xtra], axis=1)
    dst_p = jnp.concatenate([dst_p, xtra], axis=1)
    dst3 = dst_p
    src2 = jnp.stack([src_p, src_p + N])

    batch3 = batch.reshape(NB, 1, BN_BLK)

    ba1, ba2, ba3 = b1a.reshape(1, DFF), b2a.reshape(1, DFF), b3a.reshape(1, DFF)
    w1 = (W1a, ba1, bn1w.reshape(1, DFF), bn1b.reshape(1, DFF), W1b,
          b1b.reshape(1, D))
    w2 = (W2a, ba2, bn2w.reshape(1, DFF), bn2b.reshape(1, DFF), W2b,
          b2b.reshape(1, D))
    w3 = (W3a, ba3, bn3w.reshape(1, DFF), bn3b.reshape(1, DFF), W3b,
          b3b.reshape(1, D))

    pq = _first_prep(x)
    S1 = _sc_edge_sum(pq.reshape(2 * N, D), src2, dst3)
    pq, h1 = _mid_layer(S1, x, *w1)
    S2 = _sc_edge_sum(pq.reshape(2 * N, D), src2, dst3)
    pq, h2 = _mid_layer(S2, h1, *w2)
    S3 = _sc_edge_sum(pq.reshape(2 * N, D), src2, dst3)
    return _last_layer(S3, h2, *w3, batch3, Wd, bd.reshape(1, OUT))


# speedup vs baseline: 1.0803x; 1.0366x over previous
import functools

import jax
import jax.numpy as jnp
from jax import lax
from jax.experimental import pallas as pl
from jax.experimental.pallas import tpu as pltpu
from jax.experimental.pallas import tpu_sc as plsc

N = 10000
D = 128
DFF = 256
OUT = 10
G = 128
E = 320000
EPS_MSG = 1e-7
BN_EPS = 1e-5

NC = 2
NS = 16
CHUNK = 120
GRP = 6
CHG = 28
CH = CHG * GRP
EPAD = NS * CH * CHUNK
ACC_ROWS = 10128
RPT = 632
RPT_LAST = 648

BN_BLK = 2000
NB = N // BN_BLK


def _sc_edge_sum(table, src2, dst3):
    mesh = plsc.VectorSubcoreMesh(
        core_axis_name="c", subcore_axis_name="s", num_cores=NC, num_subcores=NS
    )

    @functools.partial(
        pl.kernel,
        out_type=jax.ShapeDtypeStruct((NC, ACC_ROWS, D), jnp.float32),
        mesh=mesh,
        scratch_types=[
            pltpu.VMEM_SHARED((ACC_ROWS, D), jnp.float32),
            pltpu.VMEM((GRP, CHUNK), jnp.int32),
            pltpu.VMEM((GRP, CHUNK), jnp.int32),
            pltpu.VMEM((3, CHUNK, D), jnp.float32),
            pltpu.SemaphoreType.DMA,
            pltpu.SemaphoreType.DMA,
            pltpu.SemaphoreType.DMA,
            pltpu.SemaphoreType.DMA,
        ],
    )
    def k(table_h, src_h, dst_h, out_h, acc_sh, src_v, dst_v, rows_v,
          sem_0, sem_1, sem_2, sem_d):
        c = lax.axis_index("c")
        s = lax.axis_index("s")
        sems = (sem_0, sem_1, sem_2)
        base = s * RPT

        pltpu.sync_copy(src_h.at[c, s, 0], src_v)
        pltpu.async_copy(table_h.at[src_v.at[0]], rows_v.at[0], sems[0])
        pltpu.async_copy(table_h.at[src_v.at[1]], rows_v.at[1], sems[1])
        pltpu.async_copy(dst_h.at[s, 0], dst_v, sem_d)

        zeros16 = jnp.zeros((16,), jnp.float32)

        def zrow(r, carry):
            for kk in range(D // 16):
                rows_v[2, r, pl.ds(kk * 16, 16)] = zeros16
            return carry

        lax.fori_loop(0, CHUNK, zrow, None)
        for j in range(5):
            pltpu.sync_copy(rows_v.at[2],
                            acc_sh.at[pl.ds(base + j * CHUNK, CHUNK)])

        @pl.when(s < NS - 1)
        def _():
            pltpu.sync_copy(rows_v.at[2, pl.ds(0, 32)],
                            acc_sh.at[pl.ds(base + 5 * CHUNK, 32)])

        @pl.when(s == NS - 1)
        def _():
            pltpu.sync_copy(rows_v.at[2, pl.ds(0, 48)],
                            acc_sh.at[pl.ds(base + 5 * CHUNK, 48)])

        plsc.subcore_barrier()

        def group(g, carry):
            for k in range(GRP):
                p = k % 3
                if k < GRP - 2:
                    pltpu.async_copy(table_h.at[src_v.at[k + 2]],
                                     rows_v.at[(k + 2) % 3], sems[(k + 2) % 3])
                elif k == GRP - 2:
                    pltpu.sync_copy(src_h.at[c, s, g + 1], src_v)
                    pltpu.async_copy(table_h.at[src_v.at[0]], rows_v.at[0],
                                     sems[0])
                else:
                    pltpu.async_copy(table_h.at[src_v.at[1]], rows_v.at[1],
                                     sems[1])
                pltpu.make_async_copy(table_h.at[src_v.at[k]], rows_v.at[p],
                                      sems[p]).wait()
                if k == 0:
                    pltpu.make_async_copy(dst_h.at[s, g], dst_v, sem_d).wait()
                pltpu.sync_copy(rows_v.at[p], acc_sh.at[dst_v.at[k]],
                                add=True)
                if k == GRP - 1:
                    pltpu.async_copy(dst_h.at[s, g + 1], dst_v, sem_d)
            return carry

        lax.fori_loop(0, CHG, group, None)

        pltpu.make_async_copy(table_h.at[src_v.at[0]], rows_v.at[0],
                              sems[0]).wait()
        pltpu.make_async_copy(table_h.at[src_v.at[1]], rows_v.at[1],
                              sems[1]).wait()
        pltpu.make_async_copy(dst_h.at[s, CHG], dst_v, sem_d).wait()
        plsc.subcore_barrier()

        for j in range(5):
            pltpu.sync_copy(acc_sh.at[pl.ds(base + j * CHUNK, CHUNK)],
                            rows_v.at[0])
            pltpu.sync_copy(rows_v.at[0],
                            out_h.at[c, pl.ds(base + j * CHUNK, CHUNK)])

        @pl.when(s < NS - 1)
        def _():
            pltpu.sync_copy(acc_sh.at[pl.ds(base + 5 * CHUNK, 32)],
                            rows_v.at[0, pl.ds(0, 32)])
            pltpu.sync_copy(rows_v.at[0, pl.ds(0, 32)],
                            out_h.at[c, pl.ds(base + 5 * CHUNK, 32)])

        @pl.when(s == NS - 1)
        def _():
            pltpu.sync_copy(acc_sh.at[pl.ds(base + 5 * CHUNK, 48)],
                            rows_v.at[0, pl.ds(0, 48)])
            pltpu.sync_copy(rows_v.at[0, pl.ds(0, 48)],
                            out_h.at[c, pl.ds(base + 5 * CHUNK, 48)])

    return k(table, src2, dst3)


def _prep_block(h_blk, cm):
    m = jax.nn.relu(h_blk) + EPS_MSG
    p = jnp.exp(m - (cm + EPS_MSG))
    return p, m * p


def _first_prep(x):
    def body(x_ref, o_ref, cm_scr):
        i = pl.program_id(0)
        ph = i // NB

        @pl.when(ph == 0)
        def _():
            m = jnp.max(jax.nn.relu(x_ref[...]), axis=0, keepdims=True)

            @pl.when(i == 0)
            def _():
                cm_scr[...] = m

            @pl.when(i > 0)
            def _():
                cm_scr[...] = jnp.maximum(cm_scr[...], m)

        @pl.when(ph == 1)
        def _():
            p, q = _prep_block(x_ref[...], cm_scr[...])
            o_ref[0] = p
            o_ref[1] = q

    return pl.pallas_call(
        body,
        grid=(2 * NB,),
        in_specs=[pl.BlockSpec((BN_BLK, D), lambda i: (i % NB, 0))],
        out_specs=pl.BlockSpec(
            (2, BN_BLK, D),
            lambda i: (0, jnp.where(i // NB == 1, i % NB, 0), 0)),
        out_shape=jax.ShapeDtypeStruct((2, N, D), jnp.float32),
        scratch_shapes=[pltpu.VMEM((1, D), jnp.float32)],
    )(x)


def _mlp_common(s_ref, h_ref, wa_ref, ba_ref, w_ref, b_ref, wb_ref, bb_ref,
                t_scr, st_scr, y_scr, blk, ph):
    @pl.when(ph == 0)
    def _():
        den = s_ref[0]
        num = s_ref[1]
        out = num / (den + 1e-30) + h_ref[...]
        t = jnp.dot(out, wa_ref[...], preferred_element_type=jnp.float32)
        t = t + ba_ref[...]
        t_scr[blk] = t

        @pl.when(blk == 0)
        def _():
            st_scr[...] = jnp.zeros_like(st_scr)

        st_scr[...] += jnp.concatenate(
            [jnp.sum(t, axis=0, keepdims=True),
             jnp.sum(t * t, axis=0, keepdims=True)], axis=0)

    @pl.when(ph == 1)
    def _():
        mu = st_scr[0:1] * (1.0 / N)
        var = jnp.maximum(st_scr[1:2] * (1.0 / N) - mu * mu, 0.0)
        inv = lax.rsqrt(var + BN_EPS)
        scale = inv * w_ref[...]
        shift = b_ref[...] - mu * scale
        hmid = jax.nn.relu(t_scr[blk] * scale + shift)
        y = jnp.dot(hmid, wb_ref[...], preferred_element_type=jnp.float32)
        y_scr[blk] = jax.nn.relu(y + bb_ref[...])


def _mid_layer(S, h, Wa, ba, bnw, bnb, Wb, bb):
    def body(s_ref, h_ref, wa_ref, ba_ref, w_ref, b_ref, wb_ref, bb_ref,
             pq_ref, ho_ref, t_scr, st_scr, y_scr, cm_scr):
        i = pl.program_id(0)
        ph = i // NB
        blk = i % NB
        _mlp_common(s_ref, h_ref, wa_ref, ba_ref, w_ref, b_ref, wb_ref,
                    bb_ref, t_scr, st_scr, y_scr, blk, ph)

        @pl.when(ph == 1)
        def _():
            y = y_scr[blk]
            ho_ref[...] = y
            m = jnp.max(y, axis=0, keepdims=True)

            @pl.when(blk == 0)
            def _():
                cm_scr[...] = m

            @pl.when(blk > 0)
            def _():
                cm_scr[...] = jnp.maximum(cm_scr[...], m)

        @pl.when(ph == 2)
        def _():
            p, q = _prep_block(y_scr[blk], cm_scr[...])
            pq_ref[0] = p
            pq_ref[1] = q

    return pl.pallas_call(
        body,
        grid=(3 * NB,),
        in_specs=[
            pl.BlockSpec((2, BN_BLK, D),
                         lambda i: (0, jnp.where(i // NB == 0, i % NB, 0), 0)),
            pl.BlockSpec((BN_BLK, D),
                         lambda i: (jnp.where(i // NB == 0, i % NB, 0), 0)),
            pl.BlockSpec((D, DFF), lambda i: (0, 0)),
            pl.BlockSpec((1, DFF), lambda i: (0, 0)),
            pl.BlockSpec((1, DFF), lambda i: (0, 0)),
            pl.BlockSpec((1, DFF), lambda i: (0, 0)),
            pl.BlockSpec((DFF, D), lambda i: (0, 0)),
            pl.BlockSpec((1, D), lambda i: (0, 0)),
        ],
        out_specs=[
            pl.BlockSpec(
                (2, BN_BLK, D),
                lambda i: (0, jnp.where(i // NB == 2, i % NB, 0), 0)),
            pl.BlockSpec(
                (BN_BLK, D),
                lambda i: (jnp.where(i // NB == 1, i % NB,
                                     jnp.where(i // NB == 2, NB - 1, 0)), 0)),
        ],
        out_shape=[
            jax.ShapeDtypeStruct((2, N, D), jnp.float32),
            jax.ShapeDtypeStruct((N, D), jnp.float32),
        ],
        scratch_shapes=[
            pltpu.VMEM((NB, BN_BLK, DFF), jnp.float32),
            pltpu.VMEM((2, DFF), jnp.float32),
            pltpu.VMEM((NB, BN_BLK, D), jnp.float32),
            pltpu.VMEM((1, D), jnp.float32),
        ],
    )(S, h, Wa, ba, bnw, bnb, Wb, bb)


def _last_layer(S, h, Wa, ba, bnw, bnb, Wb, bb, batch3, Wd, bd):
    def body(s_ref, h_ref, wa_ref, ba_ref, w_ref, b_ref, wb_ref, bb_ref,
             bt_ref, wd_ref, bd_ref, o_ref, t_scr, st_scr, y_scr, sums, cnts):
        i = pl.program_id(0)
        ph = i // NB
        blk = i % NB
        _mlp_common(s_ref, h_ref, wa_ref, ba_ref, w_ref, b_ref, wb_ref,
                    bb_ref, t_scr, st_scr, y_scr, blk, ph)

        @pl.when(ph == 2)
        def _():
            @pl.when(blk == 0)
            def _():
                sums[...] = jnp.zeros_like(sums)
                cnts[...] = jnp.zeros_like(cnts)

            ids = bt_ref[...]
            iota = lax.broadcasted_iota(jnp.int32, (1, G, BN_BLK), 1)
            oht = (ids == iota).astype(jnp.float32)[0]
            sums[...] += lax.dot_general(
                oht, y_scr[blk], (((1,), (0,)), ((), ())),
                preferred_element_type=jnp.float32)
            cnts[...] += lax.dot_general(
                oht, jnp.ones((BN_BLK, 1), jnp.float32),
                (((1,), (0,)), ((), ())), preferred_element_type=jnp.float32)

            @pl.when(blk == NB - 1)
            def _():
                pooled = sums[...] / jnp.maximum(cnts[...], 1.0)
                logits = jnp.dot(pooled, wd_ref[...],
                                 preferred_element_type=jnp.float32)
                o_ref[...] = jax.nn.sigmoid(logits + bd_ref[...])

    return pl.pallas_call(
        body,
        grid=(3 * NB,),
        in_specs=[
            pl.BlockSpec((2, BN_BLK, D),
                         lambda i: (0, jnp.where(i // NB == 0, i % NB, 0), 0)),
            pl.BlockSpec((BN_BLK, D),
                         lambda i: (jnp.where(i // NB == 0, i % NB, 0), 0)),
            pl.BlockSpec((D, DFF), lambda i: (0, 0)),
            pl.BlockSpec((1, DFF), lambda i: (0, 0)),
            pl.BlockSpec((1, DFF), lambda i: (0, 0)),
            pl.BlockSpec((1, DFF), lambda i: (0, 0)),
            pl.BlockSpec((DFF, D), lambda i: (0, 0)),
            pl.BlockSpec((1, D), lambda i: (0, 0)),
            pl.BlockSpec((1, 1, BN_BLK),
                         lambda i: (jnp.where(i // NB == 2, i % NB, 0), 0, 0)),
            pl.BlockSpec((D, OUT), lambda i: (0, 0)),
            pl.BlockSpec((1, OUT), lambda i: (0, 0)),
        ],
        out_specs=pl.BlockSpec((G, OUT), lambda i: (0, 0)),
        out_shape=jax.ShapeDtypeStruct((G, OUT), jnp.float32),
        scratch_shapes=[
            pltpu.VMEM((NB, BN_BLK, DFF), jnp.float32),
            pltpu.VMEM((2, DFF), jnp.float32),
            pltpu.VMEM((NB, BN_BLK, D), jnp.float32),
            pltpu.VMEM((G, D), jnp.float32),
            pltpu.VMEM((G, 1), jnp.float32),
        ],
    )(S, h, Wa, ba, bnw, bnb, Wb, bb, batch3, Wd, bd)


def kernel(x, edge_index, batch,
           W1a, b1a, bn1w, bn1b, W1b, b1b,
           W2a, b2a, bn2w, bn2b, W2b, b2b,
           W3a, b3a, bn3w, bn3b, W3b, b3b,
           Wd, bd):
    src = edge_index[0]
    dst = edge_index[1]
    pad = EPAD - E
    ar = jnp.arange(pad, dtype=jnp.int32)
    gpt = GRP * CHUNK
    src_p = jnp.concatenate([src, (ar * 67) % N]).reshape(NS, CHG, GRP, CHUNK)
    dst_p = jnp.concatenate([dst, N + (ar % 128)]).reshape(NS, CHG, GRP, CHUNK)
    xtra = (jnp.arange(NS * gpt, dtype=jnp.int32) * 131) % N
    xtra = xtra.reshape(NS, 1, GRP, CHUNK)
    src_p = jnp.concatenate([src_p, xtra], axis=1)
    dst_p = jnp.concatenate([dst_p, xtra], axis=1)
    dst3 = dst_p
    src2 = jnp.stack([src_p, src_p + N])

    batch3 = batch.reshape(NB, 1, BN_BLK)

    ba1, ba2, ba3 = b1a.reshape(1, DFF), b2a.reshape(1, DFF), b3a.reshape(1, DFF)
    w1 = (W1a, ba1, bn1w.reshape(1, DFF), bn1b.reshape(1, DFF), W1b,
          b1b.reshape(1, D))
    w2 = (W2a, ba2, bn2w.reshape(1, DFF), bn2b.reshape(1, DFF), W2b,
          b2b.reshape(1, D))
    w3 = (W3a, ba3, bn3w.reshape(1, DFF), bn3b.reshape(1, DFF), W3b,
          b3b.reshape(1, D))

    pq = _first_prep(x)
    S1 = _sc_edge_sum(pq.reshape(2 * N, D), src2, dst3)
    pq, h1 = _mid_layer(S1, x, *w1)
    S2 = _sc_edge_sum(pq.reshape(2 * N, D), src2, dst3)
    pq, h2 = _mid_layer(S2, h1, *w2)
    S3 = _sc_edge_sum(pq.reshape(2 * N, D), src2, dst3)
    return _last_layer(S3, h2, *w3, batch3, Wd, bd.reshape(1, OUT))


# speedup vs baseline: 1.0929x; 1.0116x over previous
import functools

import jax
import jax.numpy as jnp
from jax import lax
from jax.experimental import pallas as pl
from jax.experimental.pallas import tpu as pltpu
from jax.experimental.pallas import tpu_sc as plsc

N = 10000
D = 128
DFF = 256
OUT = 10
G = 128
E = 320000
EPS_MSG = 1e-7
BN_EPS = 1e-5

NC = 2
NS = 16
CHUNK = 120
GRP = 6
CHG = 28
CH = CHG * GRP
EPAD = NS * CH * CHUNK
ACC_ROWS = 10128
RPT = 632
RPT_LAST = 648

BN_BLK = 5000
NB = N // BN_BLK


def _sc_edge_sum(table, src2, dst3):
    mesh = plsc.VectorSubcoreMesh(
        core_axis_name="c", subcore_axis_name="s", num_cores=NC, num_subcores=NS
    )

    @functools.partial(
        pl.kernel,
        out_type=jax.ShapeDtypeStruct((NC, ACC_ROWS, D), jnp.float32),
        mesh=mesh,
        scratch_types=[
            pltpu.VMEM_SHARED((ACC_ROWS, D), jnp.float32),
            pltpu.VMEM((GRP, CHUNK), jnp.int32),
            pltpu.VMEM((GRP, CHUNK), jnp.int32),
            pltpu.VMEM((3, CHUNK, D), jnp.float32),
            pltpu.SemaphoreType.DMA,
            pltpu.SemaphoreType.DMA,
            pltpu.SemaphoreType.DMA,
            pltpu.SemaphoreType.DMA,
        ],
    )
    def k(table_h, src_h, dst_h, out_h, acc_sh, src_v, dst_v, rows_v,
          sem_0, sem_1, sem_2, sem_d):
        c = lax.axis_index("c")
        s = lax.axis_index("s")
        sems = (sem_0, sem_1, sem_2)
        base = s * RPT

        pltpu.sync_copy(src_h.at[c, s, 0], src_v)
        pltpu.async_copy(table_h.at[src_v.at[0]], rows_v.at[0], sems[0])
        pltpu.async_copy(table_h.at[src_v.at[1]], rows_v.at[1], sems[1])
        pltpu.async_copy(dst_h.at[s, 0], dst_v, sem_d)

        zeros16 = jnp.zeros((16,), jnp.float32)

        def zrow(r, carry):
            for kk in range(D // 16):
                rows_v[2, r, pl.ds(kk * 16, 16)] = zeros16
            return carry

        lax.fori_loop(0, CHUNK, zrow, None)
        for j in range(5):
            pltpu.sync_copy(rows_v.at[2],
                            acc_sh.at[pl.ds(base + j * CHUNK, CHUNK)])

        @pl.when(s < NS - 1)
        def _():
            pltpu.sync_copy(rows_v.at[2, pl.ds(0, 32)],
                            acc_sh.at[pl.ds(base + 5 * CHUNK, 32)])

        @pl.when(s == NS - 1)
        def _():
            pltpu.sync_copy(rows_v.at[2, pl.ds(0, 48)],
                            acc_sh.at[pl.ds(base + 5 * CHUNK, 48)])

        plsc.subcore_barrier()

        def group(g, carry):
            for k in range(GRP):
                p = k % 3
                if k < GRP - 2:
                    pltpu.async_copy(table_h.at[src_v.at[k + 2]],
                                     rows_v.at[(k + 2) % 3], sems[(k + 2) % 3])
                elif k == GRP - 2:
                    pltpu.sync_copy(src_h.at[c, s, g + 1], src_v)
                    pltpu.async_copy(table_h.at[src_v.at[0]], rows_v.at[0],
                                     sems[0])
                else:
                    pltpu.async_copy(table_h.at[src_v.at[1]], rows_v.at[1],
                                     sems[1])
                pltpu.make_async_copy(table_h.at[src_v.at[k]], rows_v.at[p],
                                      sems[p]).wait()
                if k == 0:
                    pltpu.make_async_copy(dst_h.at[s, g], dst_v, sem_d).wait()
                pltpu.sync_copy(rows_v.at[p], acc_sh.at[dst_v.at[k]],
                                add=True)
                if k == GRP - 1:
                    pltpu.async_copy(dst_h.at[s, g + 1], dst_v, sem_d)
            return carry

        lax.fori_loop(0, CHG, group, None)

        pltpu.make_async_copy(table_h.at[src_v.at[0]], rows_v.at[0],
                              sems[0]).wait()
        pltpu.make_async_copy(table_h.at[src_v.at[1]], rows_v.at[1],
                              sems[1]).wait()
        pltpu.make_async_copy(dst_h.at[s, CHG], dst_v, sem_d).wait()
        plsc.subcore_barrier()

        for j in range(5):
            pltpu.sync_copy(acc_sh.at[pl.ds(base + j * CHUNK, CHUNK)],
                            rows_v.at[0])
            pltpu.sync_copy(rows_v.at[0],
                            out_h.at[c, pl.ds(base + j * CHUNK, CHUNK)])

        @pl.when(s < NS - 1)
        def _():
            pltpu.sync_copy(acc_sh.at[pl.ds(base + 5 * CHUNK, 32)],
                            rows_v.at[0, pl.ds(0, 32)])
            pltpu.sync_copy(rows_v.at[0, pl.ds(0, 32)],
                            out_h.at[c, pl.ds(base + 5 * CHUNK, 32)])

        @pl.when(s == NS - 1)
        def _():
            pltpu.sync_copy(acc_sh.at[pl.ds(base + 5 * CHUNK, 48)],
                            rows_v.at[0, pl.ds(0, 48)])
            pltpu.sync_copy(rows_v.at[0, pl.ds(0, 48)],
                            out_h.at[c, pl.ds(base + 5 * CHUNK, 48)])

    return k(table, src2, dst3)


def _prep_block(h_blk, cm):
    m = jax.nn.relu(h_blk) + EPS_MSG
    p = jnp.exp(m - (cm + EPS_MSG))
    return p, m * p


def _first_prep(x):
    def body(x_ref, o_ref, cm_scr):
        i = pl.program_id(0)
        ph = i // NB

        @pl.when(ph == 0)
        def _():
            m = jnp.max(jax.nn.relu(x_ref[...]), axis=0, keepdims=True)

            @pl.when(i == 0)
            def _():
                cm_scr[...] = m

            @pl.when(i > 0)
            def _():
                cm_scr[...] = jnp.maximum(cm_scr[...], m)

        @pl.when(ph == 1)
        def _():
            p, q = _prep_block(x_ref[...], cm_scr[...])
            o_ref[0] = p
            o_ref[1] = q

    return pl.pallas_call(
        body,
        grid=(2 * NB,),
        in_specs=[pl.BlockSpec((BN_BLK, D), lambda i: (i % NB, 0))],
        out_specs=pl.BlockSpec(
            (2, BN_BLK, D),
            lambda i: (0, jnp.where(i // NB == 1, i % NB, 0), 0)),
        out_shape=jax.ShapeDtypeStruct((2, N, D), jnp.float32),
        scratch_shapes=[pltpu.VMEM((1, D), jnp.float32)],
    )(x)


def _mlp_common(s_ref, h_ref, wa_ref, ba_ref, w_ref, b_ref, wb_ref, bb_ref,
                t_scr, st_scr, y_scr, blk, ph):
    @pl.when(ph == 0)
    def _():
        den = s_ref[0]
        num = s_ref[1]
        out = num / (den + 1e-30) + h_ref[...]
        t = jnp.dot(out, wa_ref[...], preferred_element_type=jnp.float32)
        t = t + ba_ref[...]
        t_scr[blk] = t

        @pl.when(blk == 0)
        def _():
            st_scr[...] = jnp.zeros_like(st_scr)

        st_scr[...] += jnp.concatenate(
            [jnp.sum(t, axis=0, keepdims=True),
             jnp.sum(t * t, axis=0, keepdims=True)], axis=0)

    @pl.when(ph == 1)
    def _():
        mu = st_scr[0:1] * (1.0 / N)
        var = jnp.maximum(st_scr[1:2] * (1.0 / N) - mu * mu, 0.0)
        inv = lax.rsqrt(var + BN_EPS)
        scale = inv * w_ref[...]
        shift = b_ref[...] - mu * scale
        hmid = jax.nn.relu(t_scr[blk] * scale + shift)
        y = jnp.dot(hmid, wb_ref[...], preferred_element_type=jnp.float32)
        y_scr[blk] = jax.nn.relu(y + bb_ref[...])


def _mid_layer(S, h, Wa, ba, bnw, bnb, Wb, bb):
    def body(s_ref, h_ref, wa_ref, ba_ref, w_ref, b_ref, wb_ref, bb_ref,
             pq_ref, ho_ref, t_scr, st_scr, y_scr, cm_scr):
        i = pl.program_id(0)
        ph = i // NB
        blk = i % NB
        _mlp_common(s_ref, h_ref, wa_ref, ba_ref, w_ref, b_ref, wb_ref,
                    bb_ref, t_scr, st_scr, y_scr, blk, ph)

        @pl.when(ph == 1)
        def _():
            y = y_scr[blk]
            ho_ref[...] = y
            m = jnp.max(y, axis=0, keepdims=True)

            @pl.when(blk == 0)
            def _():
                cm_scr[...] = m

            @pl.when(blk > 0)
            def _():
                cm_scr[...] = jnp.maximum(cm_scr[...], m)

        @pl.when(ph == 2)
        def _():
            p, q = _prep_block(y_scr[blk], cm_scr[...])
            pq_ref[0] = p
            pq_ref[1] = q

    return pl.pallas_call(
        body,
        grid=(3 * NB,),
        in_specs=[
            pl.BlockSpec((2, BN_BLK, D),
                         lambda i: (0, jnp.where(i // NB == 0, i % NB, 0), 0)),
            pl.BlockSpec((BN_BLK, D),
                         lambda i: (jnp.where(i // NB == 0, i % NB, 0), 0)),
            pl.BlockSpec((D, DFF), lambda i: (0, 0)),
            pl.BlockSpec((1, DFF), lambda i: (0, 0)),
            pl.BlockSpec((1, DFF), lambda i: (0, 0)),
            pl.BlockSpec((1, DFF), lambda i: (0, 0)),
            pl.BlockSpec((DFF, D), lambda i: (0, 0)),
            pl.BlockSpec((1, D), lambda i: (0, 0)),
        ],
        out_specs=[
            pl.BlockSpec(
                (2, BN_BLK, D),
                lambda i: (0, jnp.where(i // NB == 2, i % NB, 0), 0)),
            pl.BlockSpec(
                (BN_BLK, D),
                lambda i: (jnp.where(i // NB == 1, i % NB,
                                     jnp.where(i // NB == 2, NB - 1, 0)), 0)),
        ],
        out_shape=[
            jax.ShapeDtypeStruct((2, N, D), jnp.float32),
            jax.ShapeDtypeStruct((N, D), jnp.float32),
        ],
        scratch_shapes=[
            pltpu.VMEM((NB, BN_BLK, DFF), jnp.float32),
            pltpu.VMEM((2, DFF), jnp.float32),
            pltpu.VMEM((NB, BN_BLK, D), jnp.float32),
            pltpu.VMEM((1, D), jnp.float32),
        ],
    )(S, h, Wa, ba, bnw, bnb, Wb, bb)


def _last_layer(S, h, Wa, ba, bnw, bnb, Wb, bb, batch3, Wd, bd):
    def body(s_ref, h_ref, wa_ref, ba_ref, w_ref, b_ref, wb_ref, bb_ref,
             bt_ref, wd_ref, bd_ref, o_ref, t_scr, st_scr, y_scr, sums, cnts):
        i = pl.program_id(0)
        ph = i // NB
        blk = i % NB
        _mlp_common(s_ref, h_ref, wa_ref, ba_ref, w_ref, b_ref, wb_ref,
                    bb_ref, t_scr, st_scr, y_scr, blk, ph)

        @pl.when(ph == 2)
        def _():
            @pl.when(blk == 0)
            def _():
                sums[...] = jnp.zeros_like(sums)
                cnts[...] = jnp.zeros_like(cnts)

            ids = bt_ref[...]
            iota = lax.broadcasted_iota(jnp.int32, (1, G, BN_BLK), 1)
            oht = (ids == iota).astype(jnp.float32)[0]
            sums[...] += lax.dot_general(
                oht, y_scr[blk], (((1,), (0,)), ((), ())),
                preferred_element_type=jnp.float32)
            cnts[...] += lax.dot_general(
                oht, jnp.ones((BN_BLK, 1), jnp.float32),
                (((1,), (0,)), ((), ())), preferred_element_type=jnp.float32)

            @pl.when(blk == NB - 1)
            def _():
                pooled = sums[...] / jnp.maximum(cnts[...], 1.0)
                logits = jnp.dot(pooled, wd_ref[...],
                                 preferred_element_type=jnp.float32)
                o_ref[...] = jax.nn.sigmoid(logits + bd_ref[...])

    return pl.pallas_call(
        body,
        grid=(3 * NB,),
        in_specs=[
            pl.BlockSpec((2, BN_BLK, D),
                         lambda i: (0, jnp.where(i // NB == 0, i % NB, 0), 0)),
            pl.BlockSpec((BN_BLK, D),
                         lambda i: (jnp.where(i // NB == 0, i % NB, 0), 0)),
            pl.BlockSpec((D, DFF), lambda i: (0, 0)),
            pl.BlockSpec((1, DFF), lambda i: (0, 0)),
            pl.BlockSpec((1, DFF), lambda i: (0, 0)),
            pl.BlockSpec((1, DFF), lambda i: (0, 0)),
            pl.BlockSpec((DFF, D), lambda i: (0, 0)),
            pl.BlockSpec((1, D), lambda i: (0, 0)),
            pl.BlockSpec((1, 1, BN_BLK),
                         lambda i: (jnp.where(i // NB == 2, i % NB, 0), 0, 0)),
            pl.BlockSpec((D, OUT), lambda i: (0, 0)),
            pl.BlockSpec((1, OUT), lambda i: (0, 0)),
        ],
        out_specs=pl.BlockSpec((G, OUT), lambda i: (0, 0)),
        out_shape=jax.ShapeDtypeStruct((G, OUT), jnp.float32),
        scratch_shapes=[
            pltpu.VMEM((NB, BN_BLK, DFF), jnp.float32),
            pltpu.VMEM((2, DFF), jnp.float32),
            pltpu.VMEM((NB, BN_BLK, D), jnp.float32),
            pltpu.VMEM((G, D), jnp.float32),
            pltpu.VMEM((G, 1), jnp.float32),
        ],
    )(S, h, Wa, ba, bnw, bnb, Wb, bb, batch3, Wd, bd)


def kernel(x, edge_index, batch,
           W1a, b1a, bn1w, bn1b, W1b, b1b,
           W2a, b2a, bn2w, bn2b, W2b, b2b,
           W3a, b3a, bn3w, bn3b, W3b, b3b,
           Wd, bd):
    src = edge_index[0]
    dst = edge_index[1]
    pad = EPAD - E
    ar = jnp.arange(pad, dtype=jnp.int32)
    gpt = GRP * CHUNK
    src_p = jnp.concatenate([src, (ar * 67) % N]).reshape(NS, CHG, GRP, CHUNK)
    dst_p = jnp.concatenate([dst, N + (ar % 128)]).reshape(NS, CHG, GRP, CHUNK)
    xtra = (jnp.arange(NS * gpt, dtype=jnp.int32) * 131) % N
    xtra = xtra.reshape(NS, 1, GRP, CHUNK)
    src_p = jnp.concatenate([src_p, xtra], axis=1)
    dst_p = jnp.concatenate([dst_p, xtra], axis=1)
    dst3 = dst_p
    src2 = jnp.stack([src_p, src_p + N])

    batch3 = batch.reshape(NB, 1, BN_BLK)

    ba1, ba2, ba3 = b1a.reshape(1, DFF), b2a.reshape(1, DFF), b3a.reshape(1, DFF)
    w1 = (W1a, ba1, bn1w.reshape(1, DFF), bn1b.reshape(1, DFF), W1b,
          b1b.reshape(1, D))
    w2 = (W2a, ba2, bn2w.reshape(1, DFF), bn2b.reshape(1, DFF), W2b,
          b2b.reshape(1, D))
    w3 = (W3a, ba3, bn3w.reshape(1, DFF), bn3b.reshape(1, DFF), W3b,
          b3b.reshape(1, D))

    pq = _first_prep(x)
    S1 = _sc_edge_sum(pq.reshape(2 * N, D), src2, dst3)
    pq, h1 = _mid_layer(S1, x, *w1)
    S2 = _sc_edge_sum(pq.reshape(2 * N, D), src2, dst3)
    pq, h2 = _mid_layer(S2, h1, *w2)
    S3 = _sc_edge_sum(pq.reshape(2 * N, D), src2, dst3)
    return _last_layer(S3, h2, *w3, batch3, Wd, bd.reshape(1, OUT))
